# Initial kernel scaffold; baseline (speedup 1.0000x reference)
import jax
import jax.numpy as jnp
from jax.experimental import pallas as pl


def kernel(input, indices):
    raise NotImplementedError("write your pallas kernel here")


# speedup vs baseline: 2.9969x; 2.9969x over previous
import functools

import jax
import jax.numpy as jnp
from jax import lax
from jax.experimental import pallas as pl
from jax.experimental.pallas import tpu as pltpu
from jax.experimental.pallas import tpu_sc as plsc

D = 128
CH = 128
NC, NS = 2, 16
NW = NC * NS


def _make_gather(n_rows: int):
    assert n_rows % (NW * CH) == 0
    per_w = n_rows // NW
    chunks = per_w // CH
    mesh = plsc.VectorSubcoreMesh(core_axis_name="c", subcore_axis_name="s")

    @functools.partial(
        pl.kernel,
        mesh=mesh,
        out_type=jax.ShapeDtypeStruct((n_rows, D), jnp.float32),
        scratch_types=[
            pltpu.VMEM((chunks, CH), jnp.int32),
            pltpu.VMEM((CH, D), jnp.float32),
            pltpu.SemaphoreType.DMA,
        ],
    )
    def gather(table_hbm, idx_hbm, out_hbm, idx_v, rows_v, sem):
        wid = lax.axis_index("s") * NC + lax.axis_index("c")
        pltpu.sync_copy(idx_hbm.at[wid], idx_v)
        base = wid * per_w

        def step(j, carry):
            pltpu.async_copy(table_hbm.at[idx_v.at[j]], rows_v, sem).wait()
            off = pl.multiple_of(base + j * CH, CH)
            pltpu.sync_copy(rows_v, out_hbm.at[pl.ds(off, CH)])
            return carry

        lax.fori_loop(0, chunks, step, 0)

    return gather


def kernel(input, indices):
    b, s = indices.shape
    n_rows = b * s
    idx = indices.reshape(NW, n_rows // (NW * CH), CH).astype(jnp.int32)
    out = _make_gather(n_rows)(input, idx)
    return out.reshape(b, s, D)


# speedup vs baseline: 3.3632x; 1.1222x over previous
import functools

import jax
import jax.numpy as jnp
from jax import lax
from jax.experimental import pallas as pl
from jax.experimental.pallas import tpu as pltpu
from jax.experimental.pallas import tpu_sc as plsc

D = 128
CH = 128
NC, NS = 2, 16
NW = NC * NS


NBUF = 4


def _make_gather(n_rows: int):
    assert n_rows % (NW * CH * NBUF) == 0
    per_w = n_rows // NW
    chunks = per_w // CH
    n_pass = chunks // NBUF
    mesh = plsc.VectorSubcoreMesh(core_axis_name="c", subcore_axis_name="s")

    @functools.partial(
        pl.kernel,
        mesh=mesh,
        out_type=jax.ShapeDtypeStruct((n_rows, D), jnp.float32),
        scratch_types=[
            pltpu.VMEM((chunks, CH), jnp.int32),
            pltpu.VMEM((NBUF, CH, D), jnp.float32),
        ]
        + [pltpu.SemaphoreType.DMA] * (2 * NBUF),
    )
    def gather(table_hbm, idx_hbm, out_hbm, idx_v, rows_v, *sems):
        gsem, wsem = sems[:NBUF], sems[NBUF:]
        wid = lax.axis_index("s") * NC + lax.axis_index("c")
        pltpu.sync_copy(idx_hbm.at[wid], idx_v)
        base = wid * per_w

        def g_start(chunk, b):
            pltpu.async_copy(table_hbm.at[idx_v.at[chunk]], rows_v.at[b], gsem[b])

        def g_wait(chunk, b):
            pltpu.make_async_copy(
                table_hbm.at[idx_v.at[chunk]], rows_v.at[b], gsem[b]
            ).wait()

        def w_copy(chunk, b, sem):
            off = pl.multiple_of(base + chunk * CH, CH)
            return pltpu.make_async_copy(
                rows_v.at[b], out_hbm.at[pl.ds(off, CH)], sem
            )

        for b in range(NBUF):
            g_start(b, b)

        def pass_body(i, carry):
            j = i * NBUF
            for b in range(NBUF):
                g_wait(j + b, b)
                w_copy(j + b, b, wsem[b]).start()

            @pl.when(i < n_pass - 1)
            def _():
                for b in range(NBUF):
                    w_copy(j + b, b, wsem[b]).wait()
                    g_start(j + NBUF + b, b)

            return carry

        lax.fori_loop(0, n_pass, pass_body, 0)
        for b in range(NBUF):
            w_copy(chunks - NBUF + b, b, wsem[b]).wait()

    return gather


def kernel(input, indices):
    b, s = indices.shape
    n_rows = b * s
    idx = indices.reshape(NW, n_rows // (NW * CH), CH).astype(jnp.int32)
    out = _make_gather(n_rows)(input, idx)
    return out.reshape(b, s, D)


# speedup vs baseline: 5.6487x; 1.6796x over previous
import functools

import jax
import jax.numpy as jnp
from jax import lax
from jax.experimental import pallas as pl
from jax.experimental.pallas import tpu as pltpu
from jax.experimental.pallas import tpu_sc as plsc

D = 128
NC, NS = 2, 16
NW = NC * NS
BB = 4
NBUF = 4


def _make_gather(b: int, s: int):
    ch = BB * s
    assert b % (NW * BB * NBUF) == 0
    b_per_w = b // NW
    chunks = b_per_w // BB
    n_pass = chunks // NBUF
    mesh = plsc.VectorSubcoreMesh(core_axis_name="c", subcore_axis_name="s")

    @functools.partial(
        pl.kernel,
        mesh=mesh,
        out_type=jax.ShapeDtypeStruct((b, s, D), jnp.float32),
        scratch_types=[
            pltpu.VMEM((chunks, ch), jnp.int32),
            pltpu.VMEM((NBUF, ch, D), jnp.float32),
        ]
        + [pltpu.SemaphoreType.DMA] * (2 * NBUF),
    )
    def gather(table_hbm, idx_hbm, out_hbm, idx_v, rows_v, *sems):
        gsem, wsem = sems[:NBUF], sems[NBUF:]
        wid = lax.axis_index("s") * NC + lax.axis_index("c")
        pltpu.sync_copy(idx_hbm.at[wid], idx_v)
        base_b = wid * b_per_w

        def g_start(chunk, buf):
            pltpu.async_copy(table_hbm.at[idx_v.at[chunk]], rows_v.at[buf], gsem[buf])

        def g_wait(chunk, buf):
            pltpu.make_async_copy(
                table_hbm.at[idx_v.at[chunk]], rows_v.at[buf], gsem[buf]
            ).wait()

        def w_copies(chunk, buf):
            b0 = base_b + chunk * BB
            return [
                pltpu.make_async_copy(
                    rows_v.at[buf, pl.ds(k * s, s)], out_hbm.at[b0 + k], wsem[buf]
                )
                for k in range(BB)
            ]

        for buf in range(NBUF):
            g_start(buf, buf)

        def pass_body(i, carry):
            j = i * NBUF
            for buf in range(NBUF):
                g_wait(j + buf, buf)
                for c in w_copies(j + buf, buf):
                    c.start()

            @pl.when(i < n_pass - 1)
            def _():
                for buf in range(NBUF):
                    for c in w_copies(j + buf, buf):
                        c.wait()
                    g_start(j + NBUF + buf, buf)

            return carry

        lax.fori_loop(0, n_pass, pass_body, 0)
        for buf in range(NBUF):
            for c in w_copies(chunks - NBUF + buf, buf):
                c.wait()

    return gather


def kernel(input, indices):
    b, s = indices.shape
    idx = indices.reshape(NW, b // (NW * BB), BB * s).astype(jnp.int32)
    return _make_gather(b, s)(input, idx)
